# baseline (speedup 1.0000x reference)
import functools

import jax
import jax.numpy as jnp
from jax import lax
from jax.experimental import pallas as pl
from jax.experimental.pallas import tpu as pltpu
from jax.experimental.pallas import tpu_sc as plsc

B = 16384
V = 1000000
D = 64
L = 16
NC, NS = 2, 16
NW = NC * NS
BPW = B // NW
ROWS = 2 * BPW
CHUNK = 256
NCHUNK = ROWS // CHUNK

_mesh = plsc.VectorSubcoreMesh(
    core_axis_name="c", subcore_axis_name="s", num_cores=NC, num_subcores=NS
)


def _emb_dot_body(x_hbm, w_hbm, out_hbm, xi_v, idx_v, off_v, rows_v, sums_v,
                  out_v, sem):
    wid = lax.axis_index("s") * NC + lax.axis_index("c")
    base = wid * ROWS
    pltpu.sync_copy(x_hbm.at[pl.ds(base, ROWS)], xi_v)

    def prep(c):
        xi = xi_v[pl.ds(c * L, L)]
        idx_v[pl.ds(c * L, L)] = lax.shift_right_logical(xi, 1)
        off_v[pl.ds(c * L, L)] = lax.shift_left(jnp.bitwise_and(xi, 1), 6)

    plsc.parallel_loop(0, ROWS // L, 1, unroll=4)(prep)

    lane = lax.iota(jnp.int32, L)

    for chunk in range(NCHUNK):
        pltpu.async_copy(
            w_hbm.at[idx_v.at[pl.ds(chunk * CHUNK, CHUNK)]], rows_v, sem
        ).wait()

        ebase = chunk * (CHUNK // 2)

        def element(i, ebase=ebase):
            offv = off_v[pl.ds(2 * (ebase + i), L)]
            offl = offv[0]
            offr = offv[1]
            acc = jnp.zeros((L,), jnp.float32)
            for k in range(D // L):
                lv = rows_v[2 * i, pl.ds(offl + k * L, L)]
                rv = rows_v[2 * i + 1, pl.ds(offr + k * L, L)]
                acc = acc + lv * rv
            sums_v[pl.ds((ebase + i) * (L + 1), L)] = acc

        plsc.parallel_loop(0, CHUNK // 2, 1, unroll=8)(element)

    def group(g):
        gb = (g * L + lane) * (L + 1)
        acc = plsc.load_gather(sums_v, [gb])
        for j in range(1, L):
            acc = acc + plsc.load_gather(sums_v, [gb + j])
        out_v[pl.ds(g * L, L)] = 1.0 / (1.0 + jnp.exp(-acc))

    plsc.parallel_loop(0, BPW // L, 1, unroll=2)(group)
    pltpu.sync_copy(out_v, out_hbm.at[pl.ds(wid * BPW, BPW)])


_emb_dot = pl.kernel(
    _emb_dot_body,
    out_type=jax.ShapeDtypeStruct((B,), jnp.float32),
    mesh=_mesh,
    scratch_types=[
        pltpu.VMEM((ROWS,), jnp.int32),
        pltpu.VMEM((ROWS,), jnp.int32),
        pltpu.VMEM((ROWS,), jnp.int32),
        pltpu.VMEM((CHUNK, 2 * D), jnp.float32),
        pltpu.VMEM((BPW * (L + 1),), jnp.float32),
        pltpu.VMEM((BPW,), jnp.float32),
        pltpu.SemaphoreType.DMA,
    ],
    compiler_params=pltpu.CompilerParams(
        needs_layout_passes=False, use_tc_tiling_on_sc=True
    ),
)


def kernel(x, W_g):
    return _emb_dot(x.reshape(-1), W_g.reshape(V // 2, 2 * D))


# speedup vs baseline: 1.4758x; 1.4758x over previous
import functools

import jax
import jax.numpy as jnp
from jax import lax
from jax.experimental import pallas as pl
from jax.experimental.pallas import tpu as pltpu
from jax.experimental.pallas import tpu_sc as plsc

B = 16384
V = 1000000
D = 64
L = 16
NC, NS = 2, 16
NW = NC * NS
BPW = B // NW
ROWS = 2 * BPW
CROWS = 512
NSEM = 16

_mesh = plsc.VectorSubcoreMesh(
    core_axis_name="c", subcore_axis_name="s", num_cores=NC, num_subcores=NS
)


def _emb_dot_body(x_hbm, w_hbm, out_hbm, xi_v, rows_v, sums_v, out_v, sems):
    wid = lax.axis_index("s") * NC + lax.axis_index("c")
    base = wid * ROWS
    pltpu.sync_copy(x_hbm.at[pl.ds(base, ROWS)], xi_v)

    for chunk in range(ROWS // CROWS):
        rbase = chunk * CROWS
        iv0 = xi_v[pl.ds(rbase, L)]
        for j in range(L):
            pltpu.make_async_copy(
                w_hbm.at[iv0[j]], rows_v.at[j], sems.at[j]
            ).start()

        def block(b, rbase=rbase):
            iv = xi_v[pl.ds(rbase + b * L, L)]
            for j in range(L):
                d = pltpu.make_async_copy(
                    w_hbm.at[iv[j]], rows_v.at[b * L + j], sems.at[j]
                )
                d.wait()
                d.start()

        plsc.parallel_loop(1, CROWS // L, 1)(block)
        for j in range(L):
            pltpu.make_async_copy(w_hbm.at[0], rows_v.at[j], sems.at[j]).wait()

        ebase = chunk * (CROWS // 2)

        def element(i, ebase=ebase):
            acc = jnp.zeros((L,), jnp.float32)
            for k in range(D // L):
                lv = rows_v[2 * i, pl.ds(k * L, L)]
                rv = rows_v[2 * i + 1, pl.ds(k * L, L)]
                acc = acc + lv * rv
            sums_v[pl.ds((ebase + i) * (L + 1), L)] = acc

        plsc.parallel_loop(0, CROWS // 2, 1, unroll=8)(element)

    lane = lax.iota(jnp.int32, L)

    def group(g):
        gb = (g * L + lane) * (L + 1)
        acc = plsc.load_gather(sums_v, [gb])
        for j in range(1, L):
            acc = acc + plsc.load_gather(sums_v, [gb + j])
        out_v[pl.ds(g * L, L)] = 1.0 / (1.0 + jnp.exp(-acc))

    plsc.parallel_loop(0, BPW // L, 1, unroll=2)(group)
    pltpu.sync_copy(out_v, out_hbm.at[pl.ds(wid * BPW, BPW)])


_emb_dot = pl.kernel(
    _emb_dot_body,
    out_type=jax.ShapeDtypeStruct((B,), jnp.float32),
    mesh=_mesh,
    scratch_types=[
        pltpu.VMEM((ROWS,), jnp.int32),
        pltpu.VMEM((CROWS, D), jnp.float32),
        pltpu.VMEM((BPW * (L + 1),), jnp.float32),
        pltpu.VMEM((BPW,), jnp.float32),
        pltpu.SemaphoreType.DMA((NSEM,)),
    ],
    compiler_params=pltpu.CompilerParams(
        needs_layout_passes=False, use_tc_tiling_on_sc=True
    ),
)


def kernel(x, W_g):
    return _emb_dot(x.reshape(-1), W_g)


# speedup vs baseline: 1.6590x; 1.1241x over previous
import functools

import jax
import jax.numpy as jnp
from jax import lax
from jax.experimental import pallas as pl
from jax.experimental.pallas import tpu as pltpu
from jax.experimental.pallas import tpu_sc as plsc

B = 16384
V = 1000000
D = 64
L = 16
NC, NS = 2, 16
NW = NC * NS
BPW = B // NW
ROWS = 2 * BPW
CROWS = 512
NSEM = 4

_mesh = plsc.VectorSubcoreMesh(
    core_axis_name="c", subcore_axis_name="s", num_cores=NC, num_subcores=NS
)


def _emb_dot_body(x_hbm, w_hbm, out_hbm, xi_v, rows_v, sums_v, out_v, sems):
    wid = lax.axis_index("s") * NC + lax.axis_index("c")
    base = wid * ROWS
    pltpu.sync_copy(x_hbm.at[pl.ds(base, ROWS)], xi_v)

    def fire(burst, rbase, s):
        iv = xi_v[pl.ds(rbase + burst * L, L)]
        sem = sems.at[s]
        for j in range(L):
            pltpu.make_async_copy(
                w_hbm.at[iv[j]], rows_v.at[burst * L + j], sem
            ).start()

    def drain(s):
        pltpu.make_async_copy(
            w_hbm.at[pl.ds(0, L), :],
            rows_v.at[pl.ds(0, L), :],
            sems.at[s],
        ).wait()

    NB = CROWS // L

    for chunk in range(ROWS // CROWS):
        rbase = chunk * CROWS
        for s in range(NSEM):
            fire(s, rbase, s)

        def steady(bb, rbase=rbase):
            for s in range(NSEM):
                drain(s)
                fire(bb * NSEM + s, rbase, s)

        plsc.parallel_loop(1, NB // NSEM, 1)(steady)
        for s in range(NSEM):
            drain(s)

        ebase = chunk * (CROWS // 2)

        def element(i, ebase=ebase):
            acc = jnp.zeros((L,), jnp.float32)
            for k in range(D // L):
                lv = rows_v[2 * i, pl.ds(k * L, L)]
                rv = rows_v[2 * i + 1, pl.ds(k * L, L)]
                acc = acc + lv * rv
            sums_v[pl.ds((ebase + i) * (L + 1), L)] = acc

        plsc.parallel_loop(0, CROWS // 2, 1, unroll=8)(element)

    lane = lax.iota(jnp.int32, L)

    def group(g):
        gb = (g * L + lane) * (L + 1)
        acc = plsc.load_gather(sums_v, [gb])
        for j in range(1, L):
            acc = acc + plsc.load_gather(sums_v, [gb + j])
        out_v[pl.ds(g * L, L)] = 1.0 / (1.0 + jnp.exp(-acc))

    plsc.parallel_loop(0, BPW // L, 1, unroll=2)(group)
    pltpu.sync_copy(out_v, out_hbm.at[pl.ds(wid * BPW, BPW)])


_emb_dot = pl.kernel(
    _emb_dot_body,
    out_type=jax.ShapeDtypeStruct((B,), jnp.float32),
    mesh=_mesh,
    scratch_types=[
        pltpu.VMEM((ROWS,), jnp.int32),
        pltpu.VMEM((CROWS, D), jnp.float32),
        pltpu.VMEM((BPW * (L + 1),), jnp.float32),
        pltpu.VMEM((BPW,), jnp.float32),
        pltpu.SemaphoreType.DMA((NSEM,)),
    ],
    compiler_params=pltpu.CompilerParams(
        needs_layout_passes=False, use_tc_tiling_on_sc=True
    ),
)


def kernel(x, W_g):
    return _emb_dot(x.reshape(-1), W_g)


# speedup vs baseline: 1.6612x; 1.0014x over previous
import functools

import jax
import jax.numpy as jnp
from jax import lax
from jax.experimental import pallas as pl
from jax.experimental.pallas import tpu as pltpu
from jax.experimental.pallas import tpu_sc as plsc

B = 16384
V = 1000000
D = 64
L = 16
NC, NS = 2, 16
NW = NC * NS
BPW = B // NW
ROWS = 2 * BPW
CROWS = 512
NSEM = 4

_mesh = plsc.VectorSubcoreMesh(
    core_axis_name="c", subcore_axis_name="s", num_cores=NC, num_subcores=NS
)


def _emb_dot_body(x_hbm, w_hbm, out_hbm, xi_v, rows_v, sums_v, out_v, sems):
    wid = lax.axis_index("s") * NC + lax.axis_index("c")
    base = wid * ROWS
    pltpu.sync_copy(x_hbm.at[pl.ds(base, ROWS)], xi_v)

    def fire(burst, s, rbase):
        iv = xi_v[pl.ds(rbase + burst * L, L)]
        sem = sems.at[s]
        for j in range(L):
            pltpu.make_async_copy(
                w_hbm.at[iv[j]], rows_v.at[burst * L + j], sem
            ).start()

    def drain(s):
        pltpu.make_async_copy(
            w_hbm.at[pl.ds(0, L), :],
            rows_v.at[pl.ds(0, L), :],
            sems.at[s],
        ).wait()

    def compute(burst, ecbase):
        ebase = ecbase + burst * (L // 2)
        for i in range(L // 2):
            r = burst * L + 2 * i
            acc = jnp.zeros((L,), jnp.float32)
            for k in range(D // L):
                lv = rows_v[r, pl.ds(k * L, L)]
                rv = rows_v[r + 1, pl.ds(k * L, L)]
                acc = acc + lv * rv
            sums_v[pl.ds((ebase + i) * (L + 1), L)] = acc

    NB = CROWS // L

    for chunk in range(ROWS // CROWS):
        rbase = chunk * CROWS
        ecbase = chunk * (CROWS // 2)
        for s in range(NSEM):
            fire(s, s, rbase)

        def steady(bb, rbase=rbase, ecbase=ecbase):
            for s in range(NSEM):
                b = bb * NSEM + s
                drain(s)
                fire(b, s, rbase)
                compute(b - NSEM, ecbase)

        plsc.parallel_loop(1, NB // NSEM, 1)(steady)
        for s in range(NSEM):
            drain(s)
            compute(NB - NSEM + s, ecbase)

    lane = lax.iota(jnp.int32, L)

    def group(g):
        gb = (g * L + lane) * (L + 1)
        acc = plsc.load_gather(sums_v, [gb])
        for j in range(1, L):
            acc = acc + plsc.load_gather(sums_v, [gb + j])
        out_v[pl.ds(g * L, L)] = 1.0 / (1.0 + jnp.exp(-acc))

    plsc.parallel_loop(0, BPW // L, 1, unroll=2)(group)
    pltpu.sync_copy(out_v, out_hbm.at[pl.ds(wid * BPW, BPW)])


_emb_dot = pl.kernel(
    _emb_dot_body,
    out_type=jax.ShapeDtypeStruct((B,), jnp.float32),
    mesh=_mesh,
    scratch_types=[
        pltpu.VMEM((ROWS,), jnp.int32),
        pltpu.VMEM((CROWS, D), jnp.float32),
        pltpu.VMEM((BPW * (L + 1),), jnp.float32),
        pltpu.VMEM((BPW,), jnp.float32),
        pltpu.SemaphoreType.DMA((NSEM,)),
    ],
    compiler_params=pltpu.CompilerParams(
        needs_layout_passes=False, use_tc_tiling_on_sc=True
    ),
)


def kernel(x, W_g):
    return _emb_dot(x.reshape(-1), W_g)


# speedup vs baseline: 1.6655x; 1.0025x over previous
import functools

import jax
import jax.numpy as jnp
from jax import lax
from jax.experimental import pallas as pl
from jax.experimental.pallas import tpu as pltpu
from jax.experimental.pallas import tpu_sc as plsc

B = 16384
V = 1000000
D = 64
L = 16
NC, NS = 2, 16
NW = NC * NS
BPW = B // NW
ROWS = 2 * BPW
CROWS = 512
NSEM = 8

_mesh = plsc.VectorSubcoreMesh(
    core_axis_name="c", subcore_axis_name="s", num_cores=NC, num_subcores=NS
)


def _emb_dot_body(x_hbm, w_hbm, out_hbm, xi_v, rows_v, sums_v, out_v, sems):
    wid = lax.axis_index("s") * NC + lax.axis_index("c")
    base = wid * ROWS
    pltpu.sync_copy(x_hbm.at[pl.ds(base, ROWS)], xi_v)

    def fire(burst, s, rbase):
        iv = xi_v[pl.ds(rbase + burst * L, L)]
        sem = sems.at[s]
        for j in range(L):
            pltpu.make_async_copy(
                w_hbm.at[iv[j]], rows_v.at[burst * L + j], sem
            ).start()

    def drain(s):
        pltpu.make_async_copy(
            w_hbm.at[pl.ds(0, L), :],
            rows_v.at[pl.ds(0, L), :],
            sems.at[s],
        ).wait()

    def compute(burst, ecbase):
        ebase = ecbase + burst * (L // 2)
        for i in range(L // 2):
            r = burst * L + 2 * i
            acc = jnp.zeros((L,), jnp.float32)
            for k in range(D // L):
                lv = rows_v[r, pl.ds(k * L, L)]
                rv = rows_v[r + 1, pl.ds(k * L, L)]
                acc = acc + lv * rv
            sums_v[pl.ds((ebase + i) * (L + 1), L)] = acc

    NB = CROWS // L

    for chunk in range(ROWS // CROWS):
        rbase = chunk * CROWS
        ecbase = chunk * (CROWS // 2)
        for s in range(NSEM):
            fire(s, s, rbase)

        def steady(bb, rbase=rbase, ecbase=ecbase):
            for s in range(NSEM):
                b = bb * NSEM + s
                drain(s)
                fire(b, s, rbase)
                compute(b - NSEM, ecbase)

        plsc.parallel_loop(1, NB // NSEM, 1)(steady)
        for s in range(NSEM):
            drain(s)
            compute(NB - NSEM + s, ecbase)

    lane = lax.iota(jnp.int32, L)

    def group(g):
        gb = (g * L + lane) * (L + 1)
        acc = plsc.load_gather(sums_v, [gb])
        for j in range(1, L):
            acc = acc + plsc.load_gather(sums_v, [gb + j])
        out_v[pl.ds(g * L, L)] = 1.0 / (1.0 + jnp.exp(-acc))

    plsc.parallel_loop(0, BPW // L, 1, unroll=2)(group)
    pltpu.sync_copy(out_v, out_hbm.at[pl.ds(wid * BPW, BPW)])


_emb_dot = pl.kernel(
    _emb_dot_body,
    out_type=jax.ShapeDtypeStruct((B,), jnp.float32),
    mesh=_mesh,
    scratch_types=[
        pltpu.VMEM((ROWS,), jnp.int32),
        pltpu.VMEM((CROWS, D), jnp.float32),
        pltpu.VMEM((BPW * (L + 1),), jnp.float32),
        pltpu.VMEM((BPW,), jnp.float32),
        pltpu.SemaphoreType.DMA((NSEM,)),
    ],
    compiler_params=pltpu.CompilerParams(
        needs_layout_passes=False, use_tc_tiling_on_sc=True
    ),
)


def kernel(x, W_g):
    return _emb_dot(x.reshape(-1), W_g)


# speedup vs baseline: 1.6677x; 1.0014x over previous
import jax
import jax.numpy as jnp
from jax import lax
from jax.experimental import pallas as pl
from jax.experimental.pallas import tpu as pltpu
from jax.experimental.pallas import tpu_sc as plsc

B = 16384
V = 1000000
D = 64
L = 16
NC, NS = 2, 16
NW = NC * NS
BPW = B // NW
ROWS = 2 * BPW
CROWS = 512
NSEM = 8

_mesh = plsc.VectorSubcoreMesh(
    core_axis_name="c", subcore_axis_name="s", num_cores=NC, num_subcores=NS
)


def _emb_dot_body(x_hbm, w_hbm, out_hbm, xi_v, rows_v, sums_v, out_v, sems):
    wid = lax.axis_index("s") * NC + lax.axis_index("c")
    base = wid * ROWS
    pltpu.sync_copy(x_hbm.at[pl.ds(base, ROWS)], xi_v)

    def fire(burst, s, rbase):
        iv = xi_v[pl.ds(rbase + burst * L, L)]
        sem = sems.at[s]
        for j in range(L):
            pltpu.make_async_copy(
                w_hbm.at[iv[j]], rows_v.at[burst * L + j], sem
            ).start()

    def drain(s):
        pltpu.make_async_copy(
            w_hbm.at[pl.ds(0, L), :],
            rows_v.at[pl.ds(0, L), :],
            sems.at[s],
        ).wait()

    def compute(burst, ecbase):
        ebase = ecbase + burst * (L // 2)
        for i in range(L // 2):
            r = burst * L + 2 * i
            acc = jnp.zeros((L,), jnp.float32)
            for k in range(D // L):
                lv = rows_v[r, pl.ds(k * L, L)]
                rv = rows_v[r + 1, pl.ds(k * L, L)]
                acc = acc + lv * rv
            sums_v[pl.ds((ebase + i) * (L + 1), L)] = acc

    NB = CROWS // L

    for chunk in range(ROWS // CROWS):
        rbase = chunk * CROWS
        ecbase = chunk * (CROWS // 2)
        for s in range(NSEM):
            fire(s, s, rbase)

        def steady(bb, rbase=rbase, ecbase=ecbase):
            for s in range(NSEM):
                b = bb * NSEM + s
                drain(s)
                fire(b, s, rbase)
                compute(b - NSEM, ecbase)

        plsc.parallel_loop(1, NB // NSEM, 1)(steady)
        for s in range(NSEM):
            drain(s)
            compute(NB - NSEM + s, ecbase)

    lane = lax.iota(jnp.int32, L)

    def group(g):
        gb = (g * L + lane) * (L + 1)
        acc = plsc.load_gather(sums_v, [gb])
        for j in range(1, L):
            acc = acc + plsc.load_gather(sums_v, [gb + j])
        out_v[pl.ds(g * L, L)] = 1.0 / (1.0 + jnp.exp(-acc))

    plsc.parallel_loop(0, BPW // L, 1, unroll=2)(group)
    pltpu.sync_copy(out_v, out_hbm.at[pl.ds(wid * BPW, BPW)])


_emb_dot = pl.kernel(
    _emb_dot_body,
    out_type=jax.ShapeDtypeStruct((B,), jnp.float32),
    mesh=_mesh,
    scratch_types=[
        pltpu.VMEM((ROWS,), jnp.int32),
        pltpu.VMEM((CROWS, D), jnp.float32),
        pltpu.VMEM((BPW * (L + 1),), jnp.float32),
        pltpu.VMEM((BPW,), jnp.float32),
        pltpu.SemaphoreType.DMA((NSEM,)),
    ],
    compiler_params=pltpu.CompilerParams(
        needs_layout_passes=False, use_tc_tiling_on_sc=True
    ),
)


def kernel(x, W_g):
    return _emb_dot(x.reshape(-1), W_g)
